# Initial kernel scaffold; baseline (speedup 1.0000x reference)
import functools
import math

import jax
import jax.numpy as jnp
import numpy as np
from jax import lax
from jax.experimental import pallas as pl
from jax.experimental.pallas import tpu as pltpu
from jax.experimental.pallas import tpu_sc as plsc

D = 512
S = 256
B = 1024
NW = 32
PPW = S // NW
NB = 64
L = 16


def _positional_encoding_np():
    positions = np.arange(S)[:, np.newaxis]
    rates = 1 / 10000 ** (np.arange(0, D, 2)[np.newaxis, :] / D)
    radians = positions * rates
    return np.concatenate([np.sin(radians), np.cos(radians)], axis=-1).astype(np.float32)


_PE = jnp.asarray(_positional_encoding_np())


def _make_sc_kernel():
    mesh = plsc.VectorSubcoreMesh(core_axis_name="c", subcore_axis_name="s")

    @functools.partial(
        pl.kernel,
        mesh=mesh,
        out_type=jax.ShapeDtypeStruct((B, S, D), jnp.float32),
        scratch_types=[
            pltpu.VMEM((PPW, B), jnp.int32),
            pltpu.VMEM((PPW, D), jnp.float32),
            pltpu.VMEM((NB, D), jnp.float32),
            pltpu.SemaphoreType.DMA,
        ],
    )
    def emb(enc_t_hbm, table_hbm, pe_hbm, out_hbm, idx_all, pe_v, rows_v, sem):
        cid = lax.axis_index("c")
        sid = lax.axis_index("s")
        wid = sid * 2 + cid
        p0 = wid * PPW

        pltpu.sync_copy(enc_t_hbm.at[pl.ds(p0, PPW), :], idx_all)
        pltpu.sync_copy(pe_hbm.at[pl.ds(p0, PPW), :], pe_v)

        def jloop(j, _):
            pe_rows = [pe_v[j, pl.ds(k * L, L)] for k in range(D // L)]

            def bloop(bc, _):
                b0 = bc * NB
                pltpu.async_copy(
                    table_hbm.at[idx_all.at[j, pl.ds(b0, NB)]], rows_v, sem
                ).wait()

                def add_row(b, _):
                    for k in range(D // L):
                        rows_v[b, pl.ds(k * L, L)] = (
                            rows_v[b, pl.ds(k * L, L)] + pe_rows[k]
                        )
                    return 0

                lax.fori_loop(0, NB, add_row, 0)
                pltpu.sync_copy(rows_v, out_hbm.at[pl.ds(b0, NB), p0 + j, :])
                return 0

            lax.fori_loop(0, B // NB, bloop, 0)
            return 0

        lax.fori_loop(0, PPW, jloop, 0)

    return emb


_emb = _make_sc_kernel()


def kernel(encoding, table):
    enc_t = encoding.T
    return _emb(enc_t, table, _PE)


# speedup vs baseline: 1.8463x; 1.8463x over previous
import functools

import jax
import jax.numpy as jnp
import numpy as np
from jax import lax
from jax.experimental import pallas as pl
from jax.experimental.pallas import tpu as pltpu
from jax.experimental.pallas import tpu_sc as plsc

D = 512
S = 256
B = 1024
NW = 32
PPW = S // NW
NB = 64
CPP = B // NB
NCH = PPW * CPP
NBUF = 3
L = 16


def _positional_encoding_np():
    positions = np.arange(S)[:, np.newaxis]
    rates = 1 / 10000 ** (np.arange(0, D, 2)[np.newaxis, :] / D)
    radians = positions * rates
    return np.concatenate([np.sin(radians), np.cos(radians)], axis=-1).astype(np.float32)


_PE = _positional_encoding_np()


def _make_sc_kernel():
    mesh = plsc.VectorSubcoreMesh(core_axis_name="c", subcore_axis_name="s")

    @functools.partial(
        pl.kernel,
        mesh=mesh,
        out_type=jax.ShapeDtypeStruct((B, S, D), jnp.float32),
        scratch_types=[
            pltpu.VMEM((PPW, B), jnp.int32),
            pltpu.VMEM((PPW, D), jnp.float32),
            pltpu.VMEM((NB, D), jnp.float32),
            pltpu.VMEM((NB, D), jnp.float32),
            pltpu.VMEM((NB, D), jnp.float32),
            pltpu.SemaphoreType.DMA,
            pltpu.SemaphoreType.DMA,
            pltpu.SemaphoreType.DMA,
            pltpu.SemaphoreType.DMA,
            pltpu.SemaphoreType.DMA,
            pltpu.SemaphoreType.DMA,
        ],
    )
    def emb(enc_t_hbm, table_hbm, pe_hbm, out_hbm, idx_all, pe_v,
            rows0, rows1, rows2, gsem0, gsem1, gsem2, wsem0, wsem1, wsem2):
        cid = lax.axis_index("c")
        sid = lax.axis_index("s")
        wid = sid * 2 + cid
        p0 = wid * PPW

        pltpu.sync_copy(enc_t_hbm.at[pl.ds(p0, PPW), :], idx_all)
        pltpu.sync_copy(pe_hbm.at[pl.ds(p0, PPW), :], pe_v)

        rows = (rows0, rows1, rows2)
        gsems = (gsem0, gsem1, gsem2)
        wsems = (wsem0, wsem1, wsem2)

        def chunk_j(c):
            return c // CPP

        def chunk_b0(c):
            return (c % CPP) * NB

        def idx_slice(c):
            return idx_all.at[chunk_j(c), pl.ds(chunk_b0(c), NB)]

        def out_slice(c):
            return out_hbm.at[pl.ds(chunk_b0(c), NB), p0 + chunk_j(c), :]

        def fire_gather(c, q):
            pltpu.async_copy(table_hbm.at[idx_slice(c)], rows[q], gsems[q])

        def wait_gather(c, q):
            pltpu.make_async_copy(table_hbm.at[idx_slice(c)], rows[q], gsems[q]).wait()

        def fire_write(c, q):
            pltpu.async_copy(rows[q], out_slice(c), wsems[q])

        def wait_write(c, q):
            pltpu.make_async_copy(rows[q], out_slice(c), wsems[q]).wait()

        def process(c, q):
            wait_gather(c, q)
            j = chunk_j(c)
            pe_rows = [pe_v[j, pl.ds(k * L, L)] for k in range(D // L)]

            def add_row(b, _):
                for k in range(D // L):
                    rows[q][b, pl.ds(k * L, L)] = (
                        rows[q][b, pl.ds(k * L, L)] + pe_rows[k]
                    )
                return 0

            lax.fori_loop(0, NB, add_row, 0)
            fire_write(c, q)

        NTRIP = (NCH + 1 + NBUF - 1) // NBUF

        def trip(c3, _):
            for q in range(NBUF):
                c = c3 * NBUF + q

                @pl.when(jnp.logical_and(c >= NBUF, c < NCH))
                def _():
                    wait_write(c - NBUF, q)

                @pl.when(c < NCH)
                def _():
                    fire_gather(c, q)

                @pl.when(jnp.logical_and(c >= 1, c <= NCH))
                def _():
                    process(c - 1, (q - 1) % NBUF)

            return 0

        lax.fori_loop(0, NTRIP, trip, 0)

        for cc in range(NCH - NBUF, NCH):
            wait_write(cc, cc % NBUF)

    return emb


_emb = _make_sc_kernel()


def kernel(encoding, table):
    enc_t = encoding.T
    return _emb(enc_t, table, jnp.asarray(_PE))
